# baseline (speedup 1.0000x reference)
import functools

import jax
import jax.numpy as jnp
from jax import lax
from jax.experimental import pallas as pl
from jax.experimental.pallas import tpu as pltpu
from jax.experimental.pallas import tpu_sc as plsc

B = 16384
D = 64
NCORES = 2
NSUB = 16
NW = NCORES * NSUB
B_PER_W = B // NW
CB = 8
IDS_PER_CHUNK = CB * 10
N_CHUNKS = B_PER_W // CB
ID_ROWS_PER_W = B_PER_W // CB


def _sc_body(table_hbm, ids_hbm, out_hbm, ids_v, rows0, rows1, ob0, ob1,
             gsem0, gsem1, osem0, osem1):
  c = lax.axis_index("c")
  s = lax.axis_index("s")
  wid = c * NSUB + s
  id_row0 = wid * ID_ROWS_PER_W
  out_row0 = wid * B_PER_W

  pltpu.sync_copy(ids_hbm.at[pl.ds(id_row0, ID_ROWS_PER_W)], ids_v)

  rows = (rows0, rows1)
  obufs = (ob0, ob1)
  gsems = (gsem0, gsem1)
  osems = (osem0, osem1)

  def fire_gather(g, p):
    pltpu.async_copy(table_hbm.at[ids_v.at[g]], rows[p], gsems[p])

  def compute_chunk(g, p):
    rv = rows[p]
    ob = obufs[p]
    for e in range(CB):
      r0 = e * 10
      for cc in range(4):
        col = pl.ds(cc * 16, 16)
        h = (rv[r0 + 0, col] + rv[r0 + 1, col] + rv[r0 + 2, col]
             + rv[r0 + 3, col] + rv[r0 + 4, col])
        a = (rv[r0 + 5, col] + rv[r0 + 6, col] + rv[r0 + 7, col]
             + rv[r0 + 8, col] + rv[r0 + 9, col])
        ob[e, pl.ds(cc * 16, 16)] = h
        ob[e, pl.ds(64 + cc * 16, 16)] = a
    pltpu.async_copy(ob, out_hbm.at[pl.ds(out_row0 + g * CB, CB)], osems[p])

  fire_gather(0, 0)
  fire_gather(1, 1)

  def step(g2, carry):
    for p in range(2):
      g = g2 * 2 + p
      pltpu.make_async_copy(table_hbm.at[ids_v.at[g]], rows[p], gsems[p]).wait()
      @pl.when(g >= 2)
      def _():
        pltpu.make_async_copy(
            obufs[p], out_hbm.at[pl.ds(out_row0 + g * CB, CB)], osems[p]).wait()
      compute_chunk(g, p)
      @pl.when(g + 2 < N_CHUNKS)
      def _():
        pltpu.async_copy(table_hbm.at[ids_v.at[g + 2]], rows[p], gsems[p])
    return carry

  lax.fori_loop(0, N_CHUNKS // 2, step, 0)

  for p in range(2):
    g = N_CHUNKS - 2 + p
    pltpu.make_async_copy(
        obufs[p], out_hbm.at[pl.ds(out_row0 + g * CB, CB)], osems[p]).wait()


@jax.jit
def _run(table, ids2d):
  mesh = plsc.VectorSubcoreMesh(core_axis_name="c", subcore_axis_name="s")
  fn = pl.kernel(
      _sc_body,
      out_type=jax.ShapeDtypeStruct((B, 2 * D), jnp.float32),
      mesh=mesh,
      compiler_params=pltpu.CompilerParams(use_tc_tiling_on_sc=False),
      scratch_types=[
          pltpu.VMEM((ID_ROWS_PER_W, IDS_PER_CHUNK), jnp.int32),
          pltpu.VMEM((IDS_PER_CHUNK, D), jnp.float32),
          pltpu.VMEM((IDS_PER_CHUNK, D), jnp.float32),
          pltpu.VMEM((CB, 2 * D), jnp.float32),
          pltpu.VMEM((CB, 2 * D), jnp.float32),
          pltpu.SemaphoreType.DMA,
          pltpu.SemaphoreType.DMA,
          pltpu.SemaphoreType.DMA,
          pltpu.SemaphoreType.DMA,
      ],
  )
  return fn(table, ids2d)


def kernel(x, player_embedding):
  ids = x[:, :, 0].astype(jnp.int32).reshape(B * 10 // IDS_PER_CHUNK,
                                             IDS_PER_CHUNK)
  return _run(player_embedding, ids)
